# baseline (speedup 1.0000x reference)
import functools

import jax
import jax.numpy as jnp
from jax import lax
from jax.experimental import pallas as pl
from jax.experimental.pallas import tpu as pltpu
from jax.experimental.pallas import tpu_sc as plsc

N = 10000
NH = 4
F = 64
L = 16
NC, NS = 2, 16
NW = NC * NS
NPW = 320
NPAD = NW * NPW
BB = 128
TROW = 128
ACC = 224
BN = 1000


def _tc_dense(xin, W, b2d, wq):
    Fin = xin.shape[1]
    grid = N // BN

    def body(x_ref, w_ref, b_ref, wq_ref, h_ref, q_ref, m_ref, macc):
        i = pl.program_id(0)
        h = jnp.dot(x_ref[...], w_ref[...].T, preferred_element_type=jnp.float32)
        h = h + b_ref[...]
        h = jnp.where(h >= 0.0, h, 0.2 * h)
        h_ref[...] = h
        q = jnp.dot(h, wq_ref[...].T, preferred_element_type=jnp.float32)
        q_ref[...] = q
        bm = jnp.max(q, axis=0, keepdims=True)

        @pl.when(i == 0)
        def _():
            macc[0:1, 0:8] = bm

        @pl.when(i > 0)
        def _():
            macc[0:1, 0:8] = jnp.maximum(macc[0:1, 0:8], bm)

        @pl.when(i == grid - 1)
        def _():
            m_ref[...] = macc[0:1, 0:8]

    return pl.pallas_call(
        body,
        grid=(grid,),
        in_specs=[
            pl.BlockSpec((BN, Fin), lambda i: (i, 0)),
            pl.BlockSpec((F, Fin), lambda i: (0, 0)),
            pl.BlockSpec((1, F), lambda i: (0, 0)),
            pl.BlockSpec((8, F), lambda i: (0, 0)),
        ],
        out_specs=[
            pl.BlockSpec((BN, F), lambda i: (i, 0)),
            pl.BlockSpec((BN, 8), lambda i: (i, 0)),
            pl.BlockSpec((1, 8), lambda i: (0, 0)),
        ],
        out_shape=[
            jax.ShapeDtypeStruct((N, F), jnp.float32),
            jax.ShapeDtypeStruct((N, 8), jnp.float32),
            jax.ShapeDtypeStruct((1, 8), jnp.float32),
        ],
        scratch_shapes=[pltpu.VMEM((8, 128), jnp.float32)],
    )(xin, W, b2d, wq)


def _tc_table(h, q, M):
    grid = N // BN

    def body(h_ref, q_ref, m_ref, t_ref):
        g = jnp.exp(q_ref[...] - m_ref[...])
        p0 = jnp.broadcast_to(g[:, 0:1], (BN, L))
        p1 = jnp.broadcast_to(g[:, 1:2], (BN, L))
        pad = jnp.zeros((BN, TROW - 2 * L - F), jnp.float32)
        t_ref[...] = jnp.concatenate([p0, p1, h_ref[...], pad], axis=1)

    return pl.pallas_call(
        body,
        grid=(grid,),
        in_specs=[
            pl.BlockSpec((BN, F), lambda i: (i, 0)),
            pl.BlockSpec((BN, 8), lambda i: (i, 0)),
            pl.BlockSpec((1, 8), lambda i: (0, 0)),
        ],
        out_specs=pl.BlockSpec((BN, TROW), lambda i: (i, 0)),
        out_shape=jax.ShapeDtypeStruct((N, TROW), jnp.float32),
    )(h, q, M)


def _sc_edge(table, dstp, soffp, est, final):
    OC = F if final else ACC
    mesh = plsc.VectorSubcoreMesh(core_axis_name="c", subcore_axis_name="s")

    @functools.partial(
        pl.kernel,
        out_type=jax.ShapeDtypeStruct((NPAD * OC,), jnp.float32),
        mesh=mesh,
        scratch_types=[
            pltpu.VMEM((2, BB), jnp.int32),
            pltpu.VMEM((2, BB, TROW), jnp.float32),
            pltpu.VMEM((2 * (BB + L),), jnp.int32),
            pltpu.VMEM((48,), jnp.int32),
            pltpu.VMEM((NPW * ACC,), jnp.float32),
            pltpu.VMEM((NPW * F if final else L,), jnp.float32),
            pltpu.SemaphoreType.DMA((2,)),
            pltpu.SemaphoreType.DMA,
        ],
    )
    def k(table_hbm, dst_hbm, soff_hbm, est_hbm, out_hbm,
          idx_v, stage_v, soff_v, est_v, outb, outc, sem, isem):
        wid = lax.axis_index("c") * NS + lax.axis_index("s")
        n0 = wid * NPW
        pltpu.sync_copy(est_hbm, est_v)
        e0 = est_v[pl.ds(wid, L)][0]
        e1 = est_v[pl.ds(wid + 1, L)][0]
        e0a = (e0 // 8) * 8
        joff = e0 - e0a
        cnt = e1 - e0a
        nb = lax.div(cnt + BB - 1, BB)
        zero = jnp.zeros((L,), jnp.float32)
        io = lax.iota(jnp.int32, L)

        def zrow(r, c):
            for qq in range(ACC // L):
                outb[pl.ds(r * ACC + qq * L, L)] = zero
            return c
        lax.fori_loop(0, NPW, zrow, 0)

        def idesc(b):
            slot = lax.rem(b, 2)
            eb = e0a + b * BB
            return (
                pltpu.make_async_copy(
                    dst_hbm.at[pl.ds(eb, BB)], idx_v.at[slot], isem),
                pltpu.make_async_copy(
                    soff_hbm.at[pl.ds(eb, BB)],
                    soff_v.at[pl.ds(slot * (BB + L), BB)], isem),
            )

        def gdesc(b):
            slot = lax.rem(b, 2)
            return pltpu.make_async_copy(
                table_hbm.at[idx_v.at[slot]], stage_v.at[slot], sem.at[slot])

        @pl.when(nb > 0)
        def _():
            d0, d1 = idesc(0)
            d0.start(); d1.start(); d0.wait(); d1.wait()
            gdesc(0).start()

        @pl.when(nb > 1)
        def _():
            d0, d1 = idesc(1)
            d0.start(); d1.start()

        def batch(b, c):
            slot = lax.rem(b, 2)

            @pl.when(b + 1 < nb)
            def _():
                d0, d1 = idesc(b + 1)
                d0.wait(); d1.wait()
                gdesc(b + 1).start()

            gdesc(b).wait()

            js = jnp.where(b == 0, joff, 0)
            je = jnp.minimum(cnt - b * BB, BB)

            def body1(j, base):
                rg2 = stage_v[slot, j, pl.ds(0, L)]
                rg3 = stage_v[slot, j, pl.ds(L, L)]
                sv = jnp.where(io == 0, rg2,
                               jnp.where(io == 1, rg3,
                                         jnp.where(io == 2, 1.0, 0.0)))
                for qq in range(4):
                    rh = stage_v[slot, j, pl.ds(2 * L + qq * L, L)]
                    plsc.addupdate(outb.at[pl.ds(base + qq * L, L)], rh)
                    plsc.addupdate(
                        outb.at[pl.ds(base + F + qq * L, L)], rg2 * rh)
                    plsc.addupdate(
                        outb.at[pl.ds(base + 2 * F + qq * L, L)], rg3 * rh)
                plsc.addupdate(outb.at[pl.ds(base + 3 * F, L)], sv)

            U = 8
            nmain = lax.div(je - js, U)

            def edge8(k, c2):
                jb = js + k * U
                chunk = soff_v[pl.ds(slot * (BB + L) + jb, L)]
                for u in range(U):
                    body1(jb + u, chunk[u])
                return c2

            lax.fori_loop(0, nmain, edge8, c)

            def edge1(j, c2):
                base = soff_v[pl.ds(slot * (BB + L) + j, L)][0]
                body1(j, base)
                return c2

            lax.fori_loop(js + nmain * U, je, edge1, c)

            @pl.when(b + 2 < nb)
            def _():
                d0, d1 = idesc(b + 2)
                d0.start(); d1.start()
            return c

        lax.fori_loop(0, nb, batch, 0)

        lane0 = jnp.zeros((L,), jnp.int32)
        lane1 = jnp.full((L,), 1, jnp.int32)
        lane2 = jnp.full((L,), 2, jnp.int32)

        def nrow(r, c):
            base = r * ACC
            sv = outb[pl.ds(base + 3 * F, L)]
            s2 = jnp.maximum(sv.at[lane0].get(mode="promise_in_bounds"),
                             1e-30)
            s3 = jnp.maximum(sv.at[lane1].get(mode="promise_in_bounds"),
                             1e-30)
            dg = jnp.maximum(sv.at[lane2].get(mode="promise_in_bounds"),
                             1.0)
            if final:
                for qq in range(4):
                    au = outb[pl.ds(base + qq * L, L)]
                    a2 = outb[pl.ds(base + F + qq * L, L)]
                    a3 = outb[pl.ds(base + 2 * F + qq * L, L)]
                    v = (2.0 * au / dg + a2 / s2 + a3 / s3) * 0.25
                    outc[pl.ds(r * F + qq * L, L)] = jnp.maximum(v, 0.0)
            else:
                for qq in range(4):
                    o0 = base + qq * L
                    outb[pl.ds(o0, L)] = jnp.maximum(
                        outb[pl.ds(o0, L)] / dg, 0.0)
                    outb[pl.ds(o0 + F, L)] = jnp.maximum(
                        outb[pl.ds(o0 + F, L)] / s2, 0.0)
                    outb[pl.ds(o0 + 2 * F, L)] = jnp.maximum(
                        outb[pl.ds(o0 + 2 * F, L)] / s3, 0.0)
            return c
        lax.fori_loop(0, NPW, nrow, 0)

        if final:
            pltpu.sync_copy(outc, out_hbm.at[pl.ds(n0 * F, NPW * F)])
        else:
            pltpu.sync_copy(outb, out_hbm.at[pl.ds(n0 * ACC, NPW * ACC)])

    return k(table, dstp, soffp, est)


def _merge_heads(W):
    Wr = W.reshape(-1, F, NH)
    Wm = jnp.concatenate(
        [Wr[:, :, 0] + Wr[:, :, 1], Wr[:, :, 2], Wr[:, :, 3]], axis=1)
    return jnp.pad(Wm, ((0, 0), (0, ACC - 3 * F)))


def _wq(Aw):
    w = Aw[2:4, :F] + Aw[2:4, F:]
    return jnp.pad(w, ((0, 6), (0, 0)))


def kernel(x, edge_index, W0, b0, W1, b1, W2, b2,
           A0w, A0b, A1w, A1b, A2w, A2b):
    src = edge_index[0].astype(jnp.int32)
    dst = edge_index[1].astype(jnp.int32)
    E = src.shape[0]
    bounds = jnp.arange(NW + 1, dtype=jnp.int32) * NPW
    est = jnp.searchsorted(src, bounds, side="left").astype(jnp.int32)
    est = jnp.pad(est, (0, 48 - (NW + 1)), constant_values=E)
    dstp = jnp.pad(dst, (0, BB))
    soffp = jnp.pad((src % NPW) * ACC, (0, BB))

    layers = (
        (W0, b0, A0w, False),
        (_merge_heads(W1), b1, A1w, False),
        (_merge_heads(W2), b2, A2w, True),
    )
    xin = x
    for W, b, Aw, final in layers:
        h, q, M = _tc_dense(xin, W, b.reshape(1, F), _wq(Aw))
        T = _tc_table(h, q, M)
        oc = F if final else ACC
        xin = _sc_edge(T, dstp, soffp, est, final).reshape(NPAD, oc)
    return xin[:N]


# speedup vs baseline: 14.2875x; 14.2875x over previous
import functools

import jax
import jax.numpy as jnp
from jax import lax
from jax.experimental import pallas as pl
from jax.experimental.pallas import tpu as pltpu
from jax.experimental.pallas import tpu_sc as plsc

N = 10000
NH = 4
F = 64
L = 16
NC, NS = 2, 16
NW = NC * NS
NPW = 320
NPAD = NW * NPW
BB0 = 192
TROW = 128
ACC = 224
BN = 1000


def _tc_dense(xin, W, b2d, wq):
    Fin = xin.shape[1]
    grid = N // BN

    def body(x_ref, w_ref, b_ref, wq_ref, h_ref, q_ref, m_ref, macc):
        i = pl.program_id(0)
        h = jnp.dot(x_ref[...], w_ref[...].T, preferred_element_type=jnp.float32)
        h = h + b_ref[...]
        h = jnp.where(h >= 0.0, h, 0.2 * h)
        h_ref[...] = h
        q = jnp.dot(h, wq_ref[...].T, preferred_element_type=jnp.float32)
        q_ref[...] = q
        bm = jnp.max(q, axis=0, keepdims=True)

        @pl.when(i == 0)
        def _():
            macc[0:1, 0:8] = bm

        @pl.when(i > 0)
        def _():
            macc[0:1, 0:8] = jnp.maximum(macc[0:1, 0:8], bm)

        @pl.when(i == grid - 1)
        def _():
            m_ref[...] = macc[0:1, 0:8]

    return pl.pallas_call(
        body,
        grid=(grid,),
        in_specs=[
            pl.BlockSpec((BN, Fin), lambda i: (i, 0)),
            pl.BlockSpec((F, Fin), lambda i: (0, 0)),
            pl.BlockSpec((1, F), lambda i: (0, 0)),
            pl.BlockSpec((8, F), lambda i: (0, 0)),
        ],
        out_specs=[
            pl.BlockSpec((BN, F), lambda i: (i, 0)),
            pl.BlockSpec((BN, 8), lambda i: (i, 0)),
            pl.BlockSpec((1, 8), lambda i: (0, 0)),
        ],
        out_shape=[
            jax.ShapeDtypeStruct((N, F), jnp.float32),
            jax.ShapeDtypeStruct((N, 8), jnp.float32),
            jax.ShapeDtypeStruct((1, 8), jnp.float32),
        ],
        scratch_shapes=[pltpu.VMEM((8, 128), jnp.float32)],
    )(xin, W, b2d, wq)


def _tc_table(h, q, M):
    grid = N // BN

    def body(h_ref, q_ref, m_ref, t_ref):
        g = jnp.exp(q_ref[...] - m_ref[...])
        p0 = jnp.broadcast_to(g[:, 0:1], (BN, L))
        p1 = jnp.broadcast_to(g[:, 1:2], (BN, L))
        pad = jnp.zeros((BN, TROW - 2 * L - F), jnp.float32)
        t_ref[...] = jnp.concatenate([p0, p1, h_ref[...], pad], axis=1)

    return pl.pallas_call(
        body,
        grid=(grid,),
        in_specs=[
            pl.BlockSpec((BN, F), lambda i: (i, 0)),
            pl.BlockSpec((BN, 8), lambda i: (i, 0)),
            pl.BlockSpec((1, 8), lambda i: (0, 0)),
        ],
        out_specs=pl.BlockSpec((BN, TROW), lambda i: (i, 0)),
        out_shape=jax.ShapeDtypeStruct((N, TROW), jnp.float32),
    )(h, q, M)


def _sc_edge(table, dstp, soffp, est, final):
    OC = F if final else ACC
    BB = 128 if final else BB0
    mesh = plsc.VectorSubcoreMesh(core_axis_name="c", subcore_axis_name="s")

    @functools.partial(
        pl.kernel,
        out_type=jax.ShapeDtypeStruct((NPAD * OC,), jnp.float32),
        mesh=mesh,
        scratch_types=[
            pltpu.VMEM((2 * BB,), jnp.int32),
            pltpu.VMEM((2, BB, TROW), jnp.float32),
            pltpu.VMEM((2 * (BB + L),), jnp.int32),
            pltpu.VMEM((48,), jnp.int32),
            pltpu.VMEM((NPW * ACC,), jnp.float32),
            pltpu.VMEM((NPW * F if final else L,), jnp.float32),
            pltpu.SemaphoreType.DMA((2,)),
            pltpu.SemaphoreType.DMA,
        ],
    )
    def k(table_hbm, dst_hbm, soff_hbm, est_hbm, out_hbm,
          idx_v, stage_v, soff_v, est_v, outb, outc, sem, isem):
        wid = lax.axis_index("c") * NS + lax.axis_index("s")
        n0 = wid * NPW
        pltpu.sync_copy(est_hbm, est_v)
        e0 = est_v[pl.ds(wid, L)][0]
        e1 = est_v[pl.ds(wid + 1, L)][0]
        e0a = (e0 // 8) * 8
        joff = e0 - e0a
        cnt = e1 - e0a
        nb = lax.div(cnt + BB - 1, BB)
        zero = jnp.zeros((L,), jnp.float32)
        io = lax.iota(jnp.int32, L)

        def zrow(r, c):
            for qq in range(ACC // L):
                outb[pl.ds(r * ACC + qq * L, L)] = zero
            return c
        lax.fori_loop(0, NPW, zrow, 0)

        def idesc(b):
            slot = lax.rem(b, 2)
            eb = e0a + b * BB
            return (
                pltpu.make_async_copy(
                    dst_hbm.at[pl.ds(eb, BB)],
                    idx_v.at[pl.ds(slot * BB, BB)], isem),
                pltpu.make_async_copy(
                    soff_hbm.at[pl.ds(eb, BB)],
                    soff_v.at[pl.ds(slot * (BB + L), BB)], isem),
            )

        def gdesc(b):
            slot = lax.rem(b, 2)
            ds = [pltpu.make_async_copy(
                table_hbm.at[idx_v.at[pl.ds(slot * BB, 128)]],
                stage_v.at[slot, pl.ds(0, 128)], sem.at[slot])]
            if BB > 128:
                ds.append(pltpu.make_async_copy(
                    table_hbm.at[idx_v.at[pl.ds(slot * BB + 128, BB - 128)]],
                    stage_v.at[slot, pl.ds(128, BB - 128)], sem.at[slot]))
            return ds

        @pl.when(nb > 0)
        def _():
            d0, d1 = idesc(0)
            d0.start(); d1.start(); d0.wait(); d1.wait()
            for g in gdesc(0):
                g.start()

        @pl.when(nb > 1)
        def _():
            d0, d1 = idesc(1)
            d0.start(); d1.start()

        def batch(b, c):
            slot = lax.rem(b, 2)

            @pl.when(b + 1 < nb)
            def _():
                d0, d1 = idesc(b + 1)
                d0.wait(); d1.wait()
                for g in gdesc(b + 1):
                    g.start()

            for g in gdesc(b):
                g.wait()

            js = jnp.where(b == 0, joff, 0)
            je = jnp.minimum(cnt - b * BB, BB)

            def body1(j, base):
                rg2 = stage_v[slot, j, pl.ds(0, L)]
                rg3 = stage_v[slot, j, pl.ds(L, L)]
                sv = jnp.where(io == 0, rg2,
                               jnp.where(io == 1, rg3,
                                         jnp.where(io == 2, 1.0, 0.0)))
                for qq in range(4):
                    rh = stage_v[slot, j, pl.ds(2 * L + qq * L, L)]
                    plsc.addupdate(outb.at[pl.ds(base + qq * L, L)], rh)
                    plsc.addupdate(
                        outb.at[pl.ds(base + F + qq * L, L)], rg2 * rh)
                    plsc.addupdate(
                        outb.at[pl.ds(base + 2 * F + qq * L, L)], rg3 * rh)
                plsc.addupdate(outb.at[pl.ds(base + 3 * F, L)], sv)

            U = 8
            nmain = lax.div(je - js, U)

            def edge8(k, c2):
                jb = js + k * U
                chunk = soff_v[pl.ds(slot * (BB + L) + jb, L)]
                for u in range(U):
                    body1(jb + u, chunk[u])
                return c2

            lax.fori_loop(0, nmain, edge8, c)

            def edge1(j, c2):
                base = soff_v[pl.ds(slot * (BB + L) + j, L)][0]
                body1(j, base)
                return c2

            lax.fori_loop(js + nmain * U, je, edge1, c)

            @pl.when(b + 2 < nb)
            def _():
                d0, d1 = idesc(b + 2)
                d0.start(); d1.start()
            return c

        lax.fori_loop(0, nb, batch, 0)

        lane0 = jnp.zeros((L,), jnp.int32)
        lane1 = jnp.full((L,), 1, jnp.int32)
        lane2 = jnp.full((L,), 2, jnp.int32)

        def nrow(r, c):
            base = r * ACC
            sv = outb[pl.ds(base + 3 * F, L)]
            s2 = jnp.maximum(sv.at[lane0].get(mode="promise_in_bounds"),
                             1e-30)
            s3 = jnp.maximum(sv.at[lane1].get(mode="promise_in_bounds"),
                             1e-30)
            dg = jnp.maximum(sv.at[lane2].get(mode="promise_in_bounds"),
                             1.0)
            if final:
                for qq in range(4):
                    au = outb[pl.ds(base + qq * L, L)]
                    a2 = outb[pl.ds(base + F + qq * L, L)]
                    a3 = outb[pl.ds(base + 2 * F + qq * L, L)]
                    v = (2.0 * au / dg + a2 / s2 + a3 / s3) * 0.25
                    outc[pl.ds(r * F + qq * L, L)] = jnp.maximum(v, 0.0)
            else:
                for qq in range(4):
                    o0 = base + qq * L
                    outb[pl.ds(o0, L)] = jnp.maximum(
                        outb[pl.ds(o0, L)] / dg, 0.0)
                    outb[pl.ds(o0 + F, L)] = jnp.maximum(
                        outb[pl.ds(o0 + F, L)] / s2, 0.0)
                    outb[pl.ds(o0 + 2 * F, L)] = jnp.maximum(
                        outb[pl.ds(o0 + 2 * F, L)] / s3, 0.0)
            return c
        lax.fori_loop(0, NPW, nrow, 0)

        if final:
            pltpu.sync_copy(outc, out_hbm.at[pl.ds(n0 * F, NPW * F)])
        else:
            pltpu.sync_copy(outb, out_hbm.at[pl.ds(n0 * ACC, NPW * ACC)])

    return k(table, dstp, soffp, est)


def _merge_heads(W):
    Wr = W.reshape(-1, F, NH)
    Wm = jnp.concatenate(
        [Wr[:, :, 0] + Wr[:, :, 1], Wr[:, :, 2], Wr[:, :, 3]], axis=1)
    return jnp.pad(Wm, ((0, 0), (0, ACC - 3 * F)))


def _wq(Aw):
    w = Aw[2:4, :F] + Aw[2:4, F:]
    return jnp.pad(w, ((0, 6), (0, 0)))


def kernel(x, edge_index, W0, b0, W1, b1, W2, b2,
           A0w, A0b, A1w, A1b, A2w, A2b):
    src = edge_index[0].astype(jnp.int32)
    dst = edge_index[1].astype(jnp.int32)
    E = src.shape[0]
    bounds = jnp.arange(NW + 1, dtype=jnp.int32) * NPW
    est = jnp.searchsorted(src, bounds, side="left").astype(jnp.int32)
    est = jnp.pad(est, (0, 48 - (NW + 1)), constant_values=E)
    dstp = jnp.pad(dst, (0, BB0))
    soffp = jnp.pad((src % NPW) * ACC, (0, BB0))

    layers = (
        (W0, b0, A0w, False),
        (_merge_heads(W1), b1, A1w, False),
        (_merge_heads(W2), b2, A2w, True),
    )
    xin = x
    for W, b, Aw, final in layers:
        h, q, M = _tc_dense(xin, W, b.reshape(1, F), _wq(Aw))
        T = _tc_table(h, q, M)
        oc = F if final else ACC
        xin = _sc_edge(T, dstp, soffp, est, final).reshape(NPAD, oc)
    return xin[:N]


# speedup vs baseline: 14.4918x; 1.0143x over previous
import functools

import jax
import jax.numpy as jnp
from jax import lax
from jax.experimental import pallas as pl
from jax.experimental.pallas import tpu as pltpu
from jax.experimental.pallas import tpu_sc as plsc

N = 10000
NH = 4
F = 64
L = 16
NC, NS = 2, 16
NW = NC * NS
NPW = 320
NPAD = NW * NPW
BB0 = 192
TROW = 128
ACC = 224
BN = 1000


def _tc_dense(xin, W, b2d, wq):
    Fin = xin.shape[1]
    grid = N // BN

    def body(x_ref, w_ref, b_ref, wq_ref, h_ref, q_ref, m_ref, macc):
        i = pl.program_id(0)
        h = jnp.dot(x_ref[...], w_ref[...].T, preferred_element_type=jnp.float32)
        h = h + b_ref[...]
        h = jnp.where(h >= 0.0, h, 0.2 * h)
        h_ref[...] = h
        q = jnp.dot(h, wq_ref[...].T, preferred_element_type=jnp.float32)
        q_ref[...] = q
        bm = jnp.max(q, axis=0, keepdims=True)

        @pl.when(i == 0)
        def _():
            macc[0:1, 0:8] = bm

        @pl.when(i > 0)
        def _():
            macc[0:1, 0:8] = jnp.maximum(macc[0:1, 0:8], bm)

        @pl.when(i == grid - 1)
        def _():
            m_ref[...] = macc[0:1, 0:8]

    return pl.pallas_call(
        body,
        grid=(grid,),
        in_specs=[
            pl.BlockSpec((BN, Fin), lambda i: (i, 0)),
            pl.BlockSpec((F, Fin), lambda i: (0, 0)),
            pl.BlockSpec((1, F), lambda i: (0, 0)),
            pl.BlockSpec((8, F), lambda i: (0, 0)),
        ],
        out_specs=[
            pl.BlockSpec((BN, F), lambda i: (i, 0)),
            pl.BlockSpec((BN, 8), lambda i: (i, 0)),
            pl.BlockSpec((1, 8), lambda i: (0, 0)),
        ],
        out_shape=[
            jax.ShapeDtypeStruct((N, F), jnp.float32),
            jax.ShapeDtypeStruct((N, 8), jnp.float32),
            jax.ShapeDtypeStruct((1, 8), jnp.float32),
        ],
        scratch_shapes=[pltpu.VMEM((8, 128), jnp.float32)],
    )(xin, W, b2d, wq)


def _tc_table(h, q, M):
    grid = N // BN

    def body(h_ref, q_ref, m_ref, t_ref):
        g = jnp.exp(q_ref[...] - m_ref[...])
        p0 = jnp.broadcast_to(g[:, 0:1], (BN, L))
        p1 = jnp.broadcast_to(g[:, 1:2], (BN, L))
        pad = jnp.zeros((BN, TROW - 2 * L - F), jnp.float32)
        t_ref[...] = jnp.concatenate([p0, p1, h_ref[...], pad], axis=1)

    return pl.pallas_call(
        body,
        grid=(grid,),
        in_specs=[
            pl.BlockSpec((BN, F), lambda i: (i, 0)),
            pl.BlockSpec((BN, 8), lambda i: (i, 0)),
            pl.BlockSpec((1, 8), lambda i: (0, 0)),
        ],
        out_specs=pl.BlockSpec((BN, TROW), lambda i: (i, 0)),
        out_shape=jax.ShapeDtypeStruct((N, TROW), jnp.float32),
    )(h, q, M)


def _sc_edge(table, dstp, soffp, est, final):
    OC = F if final else ACC
    BB = 128 if final else BB0
    mesh = plsc.VectorSubcoreMesh(core_axis_name="c", subcore_axis_name="s")

    @functools.partial(
        pl.kernel,
        out_type=jax.ShapeDtypeStruct((NPAD * OC,), jnp.float32),
        mesh=mesh,
        scratch_types=[
            pltpu.VMEM((2 * BB,), jnp.int32),
            pltpu.VMEM((2, BB, TROW), jnp.float32),
            pltpu.VMEM((2 * (BB + L),), jnp.int32),
            pltpu.VMEM((48,), jnp.int32),
            pltpu.VMEM((NPW * ACC,), jnp.float32),
            pltpu.VMEM((NPW * F if final else L,), jnp.float32),
            pltpu.SemaphoreType.DMA((2,)),
            pltpu.SemaphoreType.DMA,
        ],
    )
    def k(table_hbm, dst_hbm, soff_hbm, est_hbm, out_hbm,
          idx_v, stage_v, soff_v, est_v, outb, outc, sem, isem):
        wid = lax.axis_index("c") * NS + lax.axis_index("s")
        n0 = wid * NPW
        pltpu.sync_copy(est_hbm, est_v)
        e0 = est_v[pl.ds(wid, L)][0]
        e1 = est_v[pl.ds(wid + 1, L)][0]
        e0a = (e0 // 8) * 8
        joff = e0 - e0a
        cnt = e1 - e0a
        nb = lax.div(cnt + BB - 1, BB)
        zero = jnp.zeros((L,), jnp.float32)
        io = lax.iota(jnp.int32, L)

        def zrow(r, c):
            for qq in range(ACC // L):
                outb[pl.ds(r * ACC + qq * L, L)] = zero
            return c
        lax.fori_loop(0, NPW, zrow, 0)

        def idesc(b):
            slot = lax.rem(b, 2)
            eb = e0a + b * BB
            return (
                pltpu.make_async_copy(
                    dst_hbm.at[pl.ds(eb, BB)],
                    idx_v.at[pl.ds(slot * BB, BB)], isem),
                pltpu.make_async_copy(
                    soff_hbm.at[pl.ds(eb, BB)],
                    soff_v.at[pl.ds(slot * (BB + L), BB)], isem),
            )

        def gdesc(b):
            slot = lax.rem(b, 2)
            ds = [pltpu.make_async_copy(
                table_hbm.at[idx_v.at[pl.ds(slot * BB, 128)]],
                stage_v.at[slot, pl.ds(0, 128)], sem.at[slot])]
            if BB > 128:
                ds.append(pltpu.make_async_copy(
                    table_hbm.at[idx_v.at[pl.ds(slot * BB + 128, BB - 128)]],
                    stage_v.at[slot, pl.ds(128, BB - 128)], sem.at[slot]))
            return ds

        @pl.when(nb > 0)
        def _():
            d0, d1 = idesc(0)
            d0.start(); d1.start(); d0.wait(); d1.wait()
            for g in gdesc(0):
                g.start()

        @pl.when(nb > 1)
        def _():
            d0, d1 = idesc(1)
            d0.start(); d1.start()

        def batch(b, c):
            slot = lax.rem(b, 2)

            @pl.when(b + 1 < nb)
            def _():
                d0, d1 = idesc(b + 1)
                d0.wait(); d1.wait()
                for g in gdesc(b + 1):
                    g.start()

            for g in gdesc(b):
                g.wait()

            js = jnp.where(b == 0, joff, 0)
            je = jnp.minimum(cnt - b * BB, BB)

            def body1(j, base):
                rg2 = stage_v[slot, j, pl.ds(0, L)]
                rg3 = stage_v[slot, j, pl.ds(L, L)]
                sv = jnp.where(io == 0, rg2,
                               jnp.where(io == 1, rg3,
                                         jnp.where(io == 2, 1.0, 0.0)))
                for qq in range(4):
                    rh = stage_v[slot, j, pl.ds(2 * L + qq * L, L)]
                    plsc.addupdate(outb.at[pl.ds(base + qq * L, L)], rh)
                    plsc.addupdate(
                        outb.at[pl.ds(base + F + qq * L, L)], rg2 * rh)
                    plsc.addupdate(
                        outb.at[pl.ds(base + 2 * F + qq * L, L)], rg3 * rh)
                plsc.addupdate(outb.at[pl.ds(base + 3 * F, L)], sv)

            U = 16
            nmain = lax.div(je - js, U)

            def edge8(k, c2):
                jb = js + k * U
                chunk = soff_v[pl.ds(slot * (BB + L) + jb, L)]
                for u in range(U):
                    body1(jb + u, chunk[u])
                return c2

            lax.fori_loop(0, nmain, edge8, c)

            def edge1(j, c2):
                base = soff_v[pl.ds(slot * (BB + L) + j, L)][0]
                body1(j, base)
                return c2

            lax.fori_loop(js + nmain * U, je, edge1, c)

            @pl.when(b + 2 < nb)
            def _():
                d0, d1 = idesc(b + 2)
                d0.start(); d1.start()
            return c

        lax.fori_loop(0, nb, batch, 0)

        lane0 = jnp.zeros((L,), jnp.int32)
        lane1 = jnp.full((L,), 1, jnp.int32)
        lane2 = jnp.full((L,), 2, jnp.int32)

        def nrow(r, c):
            base = r * ACC
            sv = outb[pl.ds(base + 3 * F, L)]
            s2 = jnp.maximum(sv.at[lane0].get(mode="promise_in_bounds"),
                             1e-30)
            s3 = jnp.maximum(sv.at[lane1].get(mode="promise_in_bounds"),
                             1e-30)
            dg = jnp.maximum(sv.at[lane2].get(mode="promise_in_bounds"),
                             1.0)
            if final:
                for qq in range(4):
                    au = outb[pl.ds(base + qq * L, L)]
                    a2 = outb[pl.ds(base + F + qq * L, L)]
                    a3 = outb[pl.ds(base + 2 * F + qq * L, L)]
                    v = (2.0 * au / dg + a2 / s2 + a3 / s3) * 0.25
                    outc[pl.ds(r * F + qq * L, L)] = jnp.maximum(v, 0.0)
            else:
                for qq in range(4):
                    o0 = base + qq * L
                    outb[pl.ds(o0, L)] = jnp.maximum(
                        outb[pl.ds(o0, L)] / dg, 0.0)
                    outb[pl.ds(o0 + F, L)] = jnp.maximum(
                        outb[pl.ds(o0 + F, L)] / s2, 0.0)
                    outb[pl.ds(o0 + 2 * F, L)] = jnp.maximum(
                        outb[pl.ds(o0 + 2 * F, L)] / s3, 0.0)
            return c
        lax.fori_loop(0, NPW, nrow, 0)

        if final:
            pltpu.sync_copy(outc, out_hbm.at[pl.ds(n0 * F, NPW * F)])
        else:
            pltpu.sync_copy(outb, out_hbm.at[pl.ds(n0 * ACC, NPW * ACC)])

    return k(table, dstp, soffp, est)


def _merge_heads(W):
    Wr = W.reshape(-1, F, NH)
    Wm = jnp.concatenate(
        [Wr[:, :, 0] + Wr[:, :, 1], Wr[:, :, 2], Wr[:, :, 3]], axis=1)
    return jnp.pad(Wm, ((0, 0), (0, ACC - 3 * F)))


def _wq(Aw):
    w = Aw[2:4, :F] + Aw[2:4, F:]
    return jnp.pad(w, ((0, 6), (0, 0)))


def kernel(x, edge_index, W0, b0, W1, b1, W2, b2,
           A0w, A0b, A1w, A1b, A2w, A2b):
    src = edge_index[0].astype(jnp.int32)
    dst = edge_index[1].astype(jnp.int32)
    E = src.shape[0]
    bounds = jnp.arange(NW + 1, dtype=jnp.int32) * NPW
    est = jnp.searchsorted(src, bounds, side="left").astype(jnp.int32)
    est = jnp.pad(est, (0, 48 - (NW + 1)), constant_values=E)
    dstp = jnp.pad(dst, (0, BB0))
    soffp = jnp.pad((src % NPW) * ACC, (0, BB0))

    layers = (
        (W0, b0, A0w, False),
        (_merge_heads(W1), b1, A1w, False),
        (_merge_heads(W2), b2, A2w, True),
    )
    xin = x
    for W, b, Aw, final in layers:
        h, q, M = _tc_dense(xin, W, b.reshape(1, F), _wq(Aw))
        T = _tc_table(h, q, M)
        oc = F if final else ACC
        xin = _sc_edge(T, dstp, soffp, est, final).reshape(NPAD, oc)
    return xin[:N]
